# baseline (speedup 1.0000x reference)
import jax
import jax.numpy as jnp
import numpy as np
from jax import lax
from jax.experimental import pallas as pl
from jax.experimental.pallas import tpu as pltpu
from jax.experimental.pallas import tpu_sc as plsc

B = 2
N = 128
PLANE = N * N
HALO = 32
SLABX = 64 + HALO
SLABW = SLABX * PLANE
JBLK = 4
CHUNK = JBLK * N
PAIRC = 2 * CHUNK
NG = CHUNK // 16
NC8 = 8 * CHUNK
NT = 128
MAGIC = np.float32(12582912.0)


def _mesh():
    return plsc.VectorSubcoreMesh(core_axis_name="c", subcore_axis_name="s")


def _axis_terms(p):
    r0 = (p + MAGIC) - MAGIC
    r1 = r0 + 1.0
    c0 = jnp.minimum(jnp.maximum(r0, 0.0), 128.0)
    c1 = jnp.minimum(jnp.maximum(r1, 0.0), 128.0)
    i0 = c0.astype(jnp.int32)
    i1 = c1.astype(jnp.int32)
    v0 = jnp.where(i0 < 128, np.float32(1.0), np.float32(0.0))
    v1 = jnp.where(i1 < 128, np.float32(1.0), np.float32(0.0))
    f0 = (c1 - p) * v0
    f1 = (p - c0) * v1
    return i0, i1, f0, f1


def _body(img_hbm, disp_hbm, lin_hbm, out_hbm,
          slab, lintab, dispb, idxb, valb, facb, outb,
          sem_d, sem_g, sem_g2, sem_o):
    c = lax.axis_index("c")
    s = lax.axis_index("s")

    pltpu.sync_copy(lin_hbm, lintab)

    x_lo = c * 32
    half0 = c * 64

    def start_disp_pair(b, u, slot):
        i_glob = half0 + u
        d0 = ((b * N + i_glob) * 3) * PLANE + (8 * s) * N
        for comp in range(3):
            pltpu.async_copy(
                disp_hbm.at[pl.ds(d0 + comp * PLANE, PAIRC)],
                dispb.at[pl.ds(slot * (3 * PAIRC) + comp * PAIRC, PAIRC)],
                sem_d)

    def drain_disp_pair(slot):
        for comp in range(3):
            pltpu.make_async_copy(
                disp_hbm.at[pl.ds(0, PAIRC)],
                dispb.at[pl.ds(slot * (3 * PAIRC) + comp * PAIRC, PAIRC)],
                sem_d).wait()

    def pass1(b, t, slot):
        i_glob = half0 + (t >> 1)
        p = t & 1
        j0 = 8 * s + p * JBLK
        dbase = ((t >> 1) & 1) * (3 * PAIRC) + p * CHUNK
        ibase = slot * NC8
        fbase = slot * (6 * CHUNK)
        ax_i = plsc.load_gather(lintab, [jnp.full((16,), i_glob, jnp.int32)])

        for jj in range(JBLK):
            ay_j = plsc.load_gather(
                lintab, [jnp.full((16,), j0 + jj, jnp.int32)])

            @plsc.parallel_loop(jj * 8, jj * 8 + 8, unroll=2)
            def _p1(g, ay_j=ay_j):
                sl = g * 16
                dx = dispb[pl.ds(dbase + sl, 16)]
                dy = dispb[pl.ds(dbase + PAIRC + sl, 16)]
                dz = dispb[pl.ds(dbase + 2 * PAIRC + sl, 16)]
                az_k = lintab[pl.ds((g & 7) * 16, 16)]

                px = (ax_i - dx + 1.0) * 63.5
                py = (ay_j - dy + 1.0) * 63.5
                pz = (az_k - dz + 1.0) * 63.5
                x0, x1, fx0, fx1 = _axis_terms(px)
                y0, y1, fy0, fy1 = _axis_terms(py)
                z0, z1, fz0, fz1 = _axis_terms(pz)

                sx0 = jnp.minimum(jnp.maximum(x0 - x_lo, 0), SLABX - 1)
                sx1 = jnp.minimum(jnp.maximum(x1 - x_lo, 0), SLABX - 1)
                gy0 = jnp.minimum(y0, 127)
                gy1 = jnp.minimum(y1, 127)
                gz0 = jnp.minimum(z0, 127)
                gz1 = jnp.minimum(z1, 127)

                base = (sx0 << 14) + (gy0 << 7) + gz0
                dxs = (sx1 - sx0) << 14
                dys = (gy1 - gy0) << 7
                dzs = gz1 - gz0
                i001 = base + dzs
                i010 = base + dys
                i011 = i010 + dzs
                idxb[pl.ds(ibase + sl, 16)] = base
                idxb[pl.ds(ibase + CHUNK + sl, 16)] = i001
                idxb[pl.ds(ibase + 2 * CHUNK + sl, 16)] = i010
                idxb[pl.ds(ibase + 3 * CHUNK + sl, 16)] = i011
                idxb[pl.ds(ibase + 4 * CHUNK + sl, 16)] = base + dxs
                idxb[pl.ds(ibase + 5 * CHUNK + sl, 16)] = i001 + dxs
                idxb[pl.ds(ibase + 6 * CHUNK + sl, 16)] = i010 + dxs
                idxb[pl.ds(ibase + 7 * CHUNK + sl, 16)] = i011 + dxs
                facb[pl.ds(fbase + sl, 16)] = fx0
                facb[pl.ds(fbase + CHUNK + sl, 16)] = fx1
                facb[pl.ds(fbase + 2 * CHUNK + sl, 16)] = fy0
                facb[pl.ds(fbase + 3 * CHUNK + sl, 16)] = fy1
                facb[pl.ds(fbase + 4 * CHUNK + sl, 16)] = fz0
                facb[pl.ds(fbase + 5 * CHUNK + sl, 16)] = fz1

    def start_gather(slot):
        @pl.when(slot == 0)
        def _():
            pltpu.async_copy(
                slab.at[idxb.at[pl.ds(slot * NC8, NC8)]],
                valb.at[pl.ds(slot * NC8, NC8)],
                sem_g)

        @pl.when(slot == 1)
        def _():
            pltpu.async_copy(
                slab.at[idxb.at[pl.ds(slot * NC8, NC8)]],
                valb.at[pl.ds(slot * NC8, NC8)],
                sem_g2)

    def drain_gather(slot):
        @pl.when(slot == 0)
        def _():
            pltpu.make_async_copy(
                img_hbm.at[pl.ds(0, NC8)],
                valb.at[pl.ds(slot * NC8, NC8)],
                sem_g).wait()

        @pl.when(slot == 1)
        def _():
            pltpu.make_async_copy(
                img_hbm.at[pl.ds(0, NC8)],
                valb.at[pl.ds(slot * NC8, NC8)],
                sem_g2).wait()

    def pass2(tp):
        slot = tp & 1
        vbase = slot * NC8
        fbase = slot * (6 * CHUNK)
        obase = ((tp >> 1) & 1) * PAIRC + (tp & 1) * CHUNK

        @plsc.parallel_loop(0, NG, unroll=2)
        def _p2(g):
            sl = g * 16
            v000 = valb[pl.ds(vbase + sl, 16)]
            v001 = valb[pl.ds(vbase + CHUNK + sl, 16)]
            v010 = valb[pl.ds(vbase + 2 * CHUNK + sl, 16)]
            v011 = valb[pl.ds(vbase + 3 * CHUNK + sl, 16)]
            v100 = valb[pl.ds(vbase + 4 * CHUNK + sl, 16)]
            v101 = valb[pl.ds(vbase + 5 * CHUNK + sl, 16)]
            v110 = valb[pl.ds(vbase + 6 * CHUNK + sl, 16)]
            v111 = valb[pl.ds(vbase + 7 * CHUNK + sl, 16)]
            fx0 = facb[pl.ds(fbase + sl, 16)]
            fx1 = facb[pl.ds(fbase + CHUNK + sl, 16)]
            fy0 = facb[pl.ds(fbase + 2 * CHUNK + sl, 16)]
            fy1 = facb[pl.ds(fbase + 3 * CHUNK + sl, 16)]
            fz0 = facb[pl.ds(fbase + 4 * CHUNK + sl, 16)]
            fz1 = facb[pl.ds(fbase + 5 * CHUNK + sl, 16)]
            lo = fy0 * (fz0 * v000 + fz1 * v001) + \
                fy1 * (fz0 * v010 + fz1 * v011)
            hi = fy0 * (fz0 * v100 + fz1 * v101) + \
                fy1 * (fz0 * v110 + fz1 * v111)
            outb[pl.ds(obase + sl, 16)] = fx0 * lo + fx1 * hi

    def start_out_pair(b, v):
        slot = v & 1
        o0 = ((b * N + half0 + v) * N + 8 * s) * N
        pltpu.async_copy(
            outb.at[pl.ds(slot * PAIRC, PAIRC)],
            out_hbm.at[pl.ds(o0, PAIRC)],
            sem_o)

    def drain_out_pair(slot):
        pltpu.make_async_copy(
            outb.at[pl.ds(slot * PAIRC, PAIRC)],
            out_hbm.at[pl.ds(0, PAIRC)],
            sem_o).wait()

    for b in range(B):
        plsc.subcore_barrier()
        src0 = (b * N + x_lo) * PLANE + s * (6 * PLANE)
        pltpu.sync_copy(img_hbm.at[pl.ds(src0, 6 * PLANE)],
                        slab.at[pl.ds(s * (6 * PLANE), 6 * PLANE)])
        plsc.subcore_barrier()

        start_disp_pair(b, 0, 0)

        def chunk_body(t, _, b=b):
            cur = t & 1
            u = t >> 1
            slot_d = u & 1

            @pl.when(cur == 0)
            def _():
                drain_disp_pair(slot_d)

                @pl.when(u < (NT // 2) - 1)
                def _():
                    start_disp_pair(b, u + 1, 1 - slot_d)

            pass1(b, t, cur)
            start_gather(cur)

            @pl.when((cur == 1) & (t >= 5))
            def _():
                drain_out_pair(((t - 1) >> 1) & 1)

            @pl.when(t >= 1)
            def _():
                drain_gather(1 - cur)
                pass2(t - 1)

                @pl.when((cur == 0) & (t >= 2))
                def _():
                    start_out_pair(b, (t - 2) >> 1)

            return _

        lax.fori_loop(0, NT, chunk_body, None)

        drain_gather(1)
        pass2(NT - 1)
        start_out_pair(b, (NT // 2) - 1)
        drain_out_pair(0)
        drain_out_pair(1)


@jax.jit
def kernel(Img, DispField):
    lin = jnp.linspace(-1.0, 1.0, N).astype(jnp.float32)
    disp_planar = DispField.transpose(0, 1, 4, 2, 3).reshape(-1)
    run = pl.kernel(
        _body,
        out_type=jax.ShapeDtypeStruct((B * N * N * N,), jnp.float32),
        mesh=_mesh(),
        compiler_params=pltpu.CompilerParams(needs_layout_passes=False),
        scratch_types=[
            pltpu.VMEM_SHARED((SLABW,), jnp.float32),
            pltpu.VMEM((N,), jnp.float32),
            pltpu.VMEM((2 * 3 * PAIRC,), jnp.float32),
            pltpu.VMEM((2 * NC8,), jnp.int32),
            pltpu.VMEM((2 * NC8,), jnp.float32),
            pltpu.VMEM((2 * 6 * CHUNK,), jnp.float32),
            pltpu.VMEM((2 * PAIRC,), jnp.float32),
            pltpu.SemaphoreType.DMA,
            pltpu.SemaphoreType.DMA,
            pltpu.SemaphoreType.DMA,
            pltpu.SemaphoreType.DMA,
        ],
    )
    out = run(Img.reshape(-1), disp_planar, lin)
    return out.reshape(B, N, N, N, 1)


# speedup vs baseline: 1.0933x; 1.0933x over previous
import jax
import jax.numpy as jnp
import numpy as np
from jax import lax
from jax.experimental import pallas as pl
from jax.experimental.pallas import tpu as pltpu
from jax.experimental.pallas import tpu_sc as plsc

B = 2
N = 128
PLANE = N * N
HALO = 32
SLABX = 64 + HALO
SLABW = SLABX * PLANE
JBLK = 4
CHUNK = JBLK * N
PAIRC = 2 * CHUNK
NG = CHUNK // 16
NC4 = 4 * CHUNK
NT = 128
MAGIC = np.float32(12582912.0)
HIMASK = np.int32(-65536)


def _mesh():
    return plsc.VectorSubcoreMesh(core_axis_name="c", subcore_axis_name="s")


def _axis_terms(p):
    r0 = (p + MAGIC) - MAGIC
    r1 = r0 + 1.0
    c0 = jnp.minimum(jnp.maximum(r0, 0.0), 128.0)
    c1 = jnp.minimum(jnp.maximum(r1, 0.0), 128.0)
    i0 = c0.astype(jnp.int32)
    i1 = c1.astype(jnp.int32)
    v0 = jnp.where(i0 < 128, np.float32(1.0), np.float32(0.0))
    v1 = jnp.where(i1 < 128, np.float32(1.0), np.float32(0.0))
    f0 = (c1 - p) * v0
    f1 = (p - c0) * v1
    return i0, i1, f0, f1


def _body(img_hbm, disp_hbm, lin_hbm, out_hbm,
          slab, lintab, dispb, idxb, valb, facb, outb,
          sem_d, sem_g, sem_g2, sem_o):
    c = lax.axis_index("c")
    s = lax.axis_index("s")

    pltpu.sync_copy(lin_hbm, lintab)

    x_lo = c * 32
    half0 = c * 64

    def start_disp_pair(b, u, slot):
        i_glob = half0 + u
        d0 = ((b * N + i_glob) * 3) * PLANE + (8 * s) * N
        for comp in range(3):
            pltpu.async_copy(
                disp_hbm.at[pl.ds(d0 + comp * PLANE, PAIRC)],
                dispb.at[pl.ds(slot * (3 * PAIRC) + comp * PAIRC, PAIRC)],
                sem_d)

    def drain_disp_pair(slot):
        for comp in range(3):
            pltpu.make_async_copy(
                disp_hbm.at[pl.ds(0, PAIRC)],
                dispb.at[pl.ds(slot * (3 * PAIRC) + comp * PAIRC, PAIRC)],
                sem_d).wait()

    def pass1(b, t, slot):
        i_glob = half0 + (t >> 1)
        p = t & 1
        j0 = 8 * s + p * JBLK
        dbase = ((t >> 1) & 1) * (3 * PAIRC) + p * CHUNK
        ibase = slot * NC4
        fbase = slot * (6 * CHUNK)
        ax_i = plsc.load_gather(lintab, [jnp.full((16,), i_glob, jnp.int32)])

        @plsc.parallel_loop(0, NG, unroll=2)
        def _p1(g):
            sl = g * 16
            dx = dispb[pl.ds(dbase + sl, 16)]
            dy = dispb[pl.ds(dbase + PAIRC + sl, 16)]
            dz = dispb[pl.ds(dbase + 2 * PAIRC + sl, 16)]
            ay_j = plsc.load_gather(
                lintab, [jnp.full((16,), j0 + (g >> 3), jnp.int32)])
            az_k = lintab[pl.ds((g & 7) * 16, 16)]

            px = (ax_i - dx + 1.0) * 0.5 * 127.0
            py = (ay_j - dy + 1.0) * 0.5 * 127.0
            pz = (az_k - dz + 1.0) * 0.5 * 127.0
            x0, x1, fx0, fx1 = _axis_terms(px)
            y0, y1, fy0, fy1 = _axis_terms(py)
            z0, z1, fz0, fz1 = _axis_terms(pz)

            sx0 = jnp.minimum(jnp.maximum(x0 - x_lo, 0), SLABX - 1)
            sx1 = jnp.minimum(jnp.maximum(x1 - x_lo, 0), SLABX - 1)
            gy0 = jnp.minimum(y0, 127)
            gy1 = jnp.minimum(y1, 127)
            gz0 = jnp.minimum(z0, 127)
            gz1 = jnp.minimum(z1, 127)

            base = (sx0 << 14) + (gy0 << 7) + gz0
            dxs = (sx1 - sx0) << 14
            dys = (gy1 - gy0) << 7
            i010 = base + dys
            idxb[pl.ds(ibase + sl, 16)] = base
            idxb[pl.ds(ibase + CHUNK + sl, 16)] = i010
            idxb[pl.ds(ibase + 2 * CHUNK + sl, 16)] = base + dxs
            idxb[pl.ds(ibase + 3 * CHUNK + sl, 16)] = i010 + dxs
            facb[pl.ds(fbase + sl, 16)] = fx0
            facb[pl.ds(fbase + CHUNK + sl, 16)] = fx1
            facb[pl.ds(fbase + 2 * CHUNK + sl, 16)] = fy0
            facb[pl.ds(fbase + 3 * CHUNK + sl, 16)] = fy1
            facb[pl.ds(fbase + 4 * CHUNK + sl, 16)] = fz0
            facb[pl.ds(fbase + 5 * CHUNK + sl, 16)] = fz1

    def start_gather(slot):
        @pl.when(slot == 0)
        def _():
            pltpu.async_copy(
                slab.at[idxb.at[pl.ds(slot * NC4, NC4)]],
                valb.at[pl.ds(slot * NC4, NC4)],
                sem_g)

        @pl.when(slot == 1)
        def _():
            pltpu.async_copy(
                slab.at[idxb.at[pl.ds(slot * NC4, NC4)]],
                valb.at[pl.ds(slot * NC4, NC4)],
                sem_g2)

    def drain_gather(slot):
        @pl.when(slot == 0)
        def _():
            pltpu.make_async_copy(
                img_hbm.at[pl.ds(0, NC4)],
                valb.at[pl.ds(slot * NC4, NC4)],
                sem_g).wait()

        @pl.when(slot == 1)
        def _():
            pltpu.make_async_copy(
                img_hbm.at[pl.ds(0, NC4)],
                valb.at[pl.ds(slot * NC4, NC4)],
                sem_g2).wait()

    def pass2(tp):
        slot = tp & 1
        vbase = slot * NC4
        fbase = slot * (6 * CHUNK)
        obase = ((tp >> 1) & 1) * PAIRC + (tp & 1) * CHUNK

        @plsc.parallel_loop(0, NG, unroll=2)
        def _p2(g):
            sl = g * 16
            p00 = valb[pl.ds(vbase + sl, 16)]
            p01 = valb[pl.ds(vbase + CHUNK + sl, 16)]
            p10 = valb[pl.ds(vbase + 2 * CHUNK + sl, 16)]
            p11 = valb[pl.ds(vbase + 3 * CHUNK + sl, 16)]
            v000 = lax.bitcast_convert_type(p00 << 16, jnp.float32)
            v001 = lax.bitcast_convert_type(p00 & HIMASK, jnp.float32)
            v010 = lax.bitcast_convert_type(p01 << 16, jnp.float32)
            v011 = lax.bitcast_convert_type(p01 & HIMASK, jnp.float32)
            v100 = lax.bitcast_convert_type(p10 << 16, jnp.float32)
            v101 = lax.bitcast_convert_type(p10 & HIMASK, jnp.float32)
            v110 = lax.bitcast_convert_type(p11 << 16, jnp.float32)
            v111 = lax.bitcast_convert_type(p11 & HIMASK, jnp.float32)
            fx0 = facb[pl.ds(fbase + sl, 16)]
            fx1 = facb[pl.ds(fbase + CHUNK + sl, 16)]
            fy0 = facb[pl.ds(fbase + 2 * CHUNK + sl, 16)]
            fy1 = facb[pl.ds(fbase + 3 * CHUNK + sl, 16)]
            fz0 = facb[pl.ds(fbase + 4 * CHUNK + sl, 16)]
            fz1 = facb[pl.ds(fbase + 5 * CHUNK + sl, 16)]
            lo = fy0 * (fz0 * v000 + fz1 * v001) + \
                fy1 * (fz0 * v010 + fz1 * v011)
            hi = fy0 * (fz0 * v100 + fz1 * v101) + \
                fy1 * (fz0 * v110 + fz1 * v111)
            outb[pl.ds(obase + sl, 16)] = fx0 * lo + fx1 * hi

    def start_out_pair(b, v):
        slot = v & 1
        o0 = ((b * N + half0 + v) * N + 8 * s) * N
        pltpu.async_copy(
            outb.at[pl.ds(slot * PAIRC, PAIRC)],
            out_hbm.at[pl.ds(o0, PAIRC)],
            sem_o)

    def drain_out_pair(slot):
        pltpu.make_async_copy(
            outb.at[pl.ds(slot * PAIRC, PAIRC)],
            out_hbm.at[pl.ds(0, PAIRC)],
            sem_o).wait()

    for b in range(B):
        plsc.subcore_barrier()
        src0 = (b * N + x_lo) * PLANE + s * (6 * PLANE)
        pltpu.sync_copy(img_hbm.at[pl.ds(src0, 6 * PLANE)],
                        slab.at[pl.ds(s * (6 * PLANE), 6 * PLANE)])
        plsc.subcore_barrier()

        start_disp_pair(b, 0, 0)

        def chunk_body(t, _, b=b):
            cur = t & 1
            u = t >> 1
            slot_d = u & 1

            @pl.when(cur == 0)
            def _():
                drain_disp_pair(slot_d)

                @pl.when(u < (NT // 2) - 1)
                def _():
                    start_disp_pair(b, u + 1, 1 - slot_d)

            pass1(b, t, cur)
            start_gather(cur)

            @pl.when((cur == 1) & (t >= 5))
            def _():
                drain_out_pair(((t - 1) >> 1) & 1)

            @pl.when(t >= 1)
            def _():
                drain_gather(1 - cur)
                pass2(t - 1)

                @pl.when((cur == 0) & (t >= 2))
                def _():
                    start_out_pair(b, (t - 2) >> 1)

            return _

        lax.fori_loop(0, NT, chunk_body, None)

        drain_gather(1)
        pass2(NT - 1)
        start_out_pair(b, (NT // 2) - 1)
        drain_out_pair(0)
        drain_out_pair(1)


@jax.jit
def kernel(Img, DispField):
    lin = jnp.linspace(-1.0, 1.0, N).astype(jnp.float32)
    disp_planar = DispField.transpose(0, 1, 4, 2, 3).reshape(-1)
    img = Img.reshape(B, N, N, N)
    img_next = jnp.concatenate(
        [img[..., 1:], jnp.zeros_like(img[..., :1])], axis=-1)
    img_pairs = lax.bitcast_convert_type(
        jnp.stack([img, img_next], axis=-1).astype(jnp.bfloat16),
        jnp.int32).reshape(-1)
    run = pl.kernel(
        _body,
        out_type=jax.ShapeDtypeStruct((B * N * N * N,), jnp.float32),
        mesh=_mesh(),
        compiler_params=pltpu.CompilerParams(needs_layout_passes=False),
        scratch_types=[
            pltpu.VMEM_SHARED((SLABW,), jnp.int32),
            pltpu.VMEM((N,), jnp.float32),
            pltpu.VMEM((2 * 3 * PAIRC,), jnp.float32),
            pltpu.VMEM((2 * NC4,), jnp.int32),
            pltpu.VMEM((2 * NC4,), jnp.int32),
            pltpu.VMEM((2 * 6 * CHUNK,), jnp.float32),
            pltpu.VMEM((2 * PAIRC,), jnp.float32),
            pltpu.SemaphoreType.DMA,
            pltpu.SemaphoreType.DMA,
            pltpu.SemaphoreType.DMA,
            pltpu.SemaphoreType.DMA,
        ],
    )
    out = run(img_pairs, disp_planar, lin)
    return out.reshape(B, N, N, N, 1)
